# baseline (speedup 1.0000x reference)
import functools

import jax
import jax.numpy as jnp
from jax import lax
from jax.experimental import pallas as pl
from jax.experimental.pallas import tpu as pltpu
from jax.experimental.pallas import tpu_sc as plsc

NUM_SC = 2
NUM_TILES = 16
NW = NUM_SC * NUM_TILES


def _sc_mesh():
    return plsc.VectorSubcoreMesh(core_axis_name="c", subcore_axis_name="s")


def _make_count_kernel(n, d, ch, k):
    rows_per_tile = n // NUM_TILES
    win = 8

    @functools.partial(
        pl.kernel,
        mesh=_sc_mesh(),
        out_type=jax.ShapeDtypeStruct((NUM_SC, n, d), jnp.float32),
        scratch_types=[
            pltpu.VMEM((ch, k), jnp.int32),
            pltpu.VMEM((k, d), jnp.float32),
            pltpu.VMEM_SHARED((n, d), jnp.float32),
            pltpu.SemaphoreType.DMA,
        ],
    )
    def count_kernel(dst_hbm, zero_hbm, one_hbm, out_hbm, dst_v, ones_v,
                     acc_sh, sem):
        c = lax.axis_index("c")
        s = lax.axis_index("s")
        wid = s * NUM_SC + c
        r0 = s * rows_per_tile
        pltpu.sync_copy(zero_hbm.at[pl.ds(r0, rows_per_tile)],
                        acc_sh.at[pl.ds(r0, rows_per_tile)])
        pltpu.sync_copy(dst_hbm.at[wid], dst_v)
        pltpu.sync_copy(one_hbm, ones_v)
        plsc.subcore_barrier()

        @pl.loop(0, ch)
        def _(j):
            pltpu.sync_copy(ones_v, acc_sh.at[dst_v.at[j]], add=True)

        plsc.subcore_barrier()
        pltpu.sync_copy(acc_sh.at[pl.ds(r0, rows_per_tile)],
                        out_hbm.at[c, pl.ds(r0, rows_per_tile)])

    return count_kernel


def _make_agg_kernel(n, d, ch, k):
    rows_per_tile = n // NUM_TILES

    @functools.partial(
        pl.kernel,
        mesh=_sc_mesh(),
        out_type=jax.ShapeDtypeStruct((NUM_SC, n, d), jnp.float32),
        scratch_types=[
            pltpu.VMEM((ch, k), jnp.int32),
            pltpu.VMEM((ch, k), jnp.int32),
            pltpu.VMEM((k, d), jnp.float32),
            pltpu.VMEM_SHARED((n, d), jnp.float32),
        ],
    )
    def agg_kernel(h_hbm, src_hbm, dst_hbm, zero_hbm, out_hbm,
                   src_v, dst_v, rows_v, acc_sh):
        c = lax.axis_index("c")
        s = lax.axis_index("s")
        wid = s * NUM_SC + c
        r0 = s * rows_per_tile
        pltpu.sync_copy(zero_hbm.at[pl.ds(r0, rows_per_tile)],
                        acc_sh.at[pl.ds(r0, rows_per_tile)])
        pltpu.sync_copy(src_hbm.at[wid], src_v)
        pltpu.sync_copy(dst_hbm.at[wid], dst_v)
        plsc.subcore_barrier()

        @pl.loop(0, ch)
        def _(j):
            pltpu.sync_copy(h_hbm.at[src_v.at[j]], rows_v)
            pltpu.sync_copy(rows_v, acc_sh.at[dst_v.at[j]], add=True)

        plsc.subcore_barrier()
        pltpu.sync_copy(acc_sh.at[pl.ds(r0, rows_per_tile)],
                        out_hbm.at[c, pl.ds(r0, rows_per_tile)])

    return agg_kernel


def _tc_layer_body(p_ref, cp_ref, h_ref, wl_ref, wr_ref, b_ref, g_ref, be_ref,
                   o_ref, *, bn):
    n = h_ref.shape[0]
    cnt = cp_ref[0, :n, 0:1] + cp_ref[1, :n, 0:1]
    mean = (p_ref[0, :n, :] + p_ref[1, :n, :]) / jnp.maximum(cnt, 1.0)
    z = jnp.dot(mean, wl_ref[...], preferred_element_type=jnp.float32)
    z = z + jnp.dot(h_ref[...], wr_ref[...], preferred_element_type=jnp.float32)
    z = z + b_ref[...]
    if bn:
        mu = jnp.mean(z, axis=0, keepdims=True)
        var = jnp.mean((z - mu) ** 2, axis=0, keepdims=True)
        z = (z - mu) * lax.rsqrt(var + 1e-5) * g_ref[...] + be_ref[...]
        z = jnp.maximum(z, 0.0)
    o_ref[...] = z


def _tc_layer(p, cp, h, wl, wr, b, g, be, *, bn):
    n, d = h.shape
    body = functools.partial(_tc_layer_body, bn=bn)
    return pl.pallas_call(
        body,
        out_shape=jax.ShapeDtypeStruct((n, d), jnp.float32),
    )(p, cp, h, wl, wr, b.reshape(1, d), g.reshape(1, d), be.reshape(1, d))


def kernel(x, edge_index, Wl0, Wr0, b0, Wl1, Wr1, b1, Wl2, Wr2, b2,
           g0, be0, g1, be1):
    n, d = x.shape
    e = edge_index.shape[1]

    npad = -(-n // (NUM_TILES * 8)) * (NUM_TILES * 8)

    k = 80
    ch = -(-e // (NW * k))
    ch += ch % 2
    pad = NW * ch * k - e

    src = edge_index[0].astype(jnp.int32)
    dst = edge_index[1].astype(jnp.int32)
    pad_dst = n + jax.lax.rem(jnp.arange(pad, dtype=jnp.int32),
                              jnp.int32(npad - n))
    src = jnp.concatenate([src, jnp.zeros((pad,), jnp.int32)])
    dst = jnp.concatenate([dst, pad_dst])
    src3 = src.reshape(NW, ch, k)
    dst3 = dst.reshape(NW, ch, k)
    zeros_nd = jnp.zeros((npad, d), jnp.float32)
    ones_k = jnp.ones((k, d), jnp.float32)

    count_k = _make_count_kernel(npad, d, ch, k)
    agg_k = _make_agg_kernel(npad, d, ch, k)

    cp = count_k(dst3, zeros_nd, ones_k)
    p0 = agg_k(x, src3, dst3, zeros_nd)
    h1 = _tc_layer(p0, cp, x, Wl0, Wr0, b0, g0, be0, bn=True)
    p1 = agg_k(h1, src3, dst3, zeros_nd)
    h2 = _tc_layer(p1, cp, h1, Wl1, Wr1, b1, g1, be1, bn=True)
    p2 = agg_k(h2, src3, dst3, zeros_nd)
    return _tc_layer(p2, cp, h2, Wl2, Wr2, b2, g1, be1, bn=False)


# speedup vs baseline: 1.4126x; 1.4126x over previous
import functools

import jax
import jax.numpy as jnp
from jax import lax
from jax.experimental import pallas as pl
from jax.experimental.pallas import tpu as pltpu
from jax.experimental.pallas import tpu_sc as plsc

NUM_SC = 2
NUM_TILES = 16
NW = NUM_SC * NUM_TILES


def _sc_mesh():
    return plsc.VectorSubcoreMesh(core_axis_name="c", subcore_axis_name="s")


def _make_count_kernel(n, d, ch, k):
    rows_per_tile = n // NUM_TILES
    win = 8

    @functools.partial(
        pl.kernel,
        mesh=_sc_mesh(),
        out_type=jax.ShapeDtypeStruct((NUM_SC, n, d), jnp.float32),
        scratch_types=[
            pltpu.VMEM((ch, k), jnp.int32),
            pltpu.VMEM((k, d), jnp.float32),
            pltpu.VMEM_SHARED((n, d), jnp.float32),
            pltpu.SemaphoreType.DMA,
        ],
    )
    def count_kernel(dst_hbm, zero_hbm, one_hbm, out_hbm, dst_v, ones_v,
                     acc_sh, sem):
        c = lax.axis_index("c")
        s = lax.axis_index("s")
        wid = s * NUM_SC + c
        r0 = s * rows_per_tile
        pltpu.sync_copy(zero_hbm.at[pl.ds(r0, rows_per_tile)],
                        acc_sh.at[pl.ds(r0, rows_per_tile)])
        pltpu.sync_copy(dst_hbm.at[wid], dst_v)
        pltpu.sync_copy(one_hbm, ones_v)
        plsc.subcore_barrier()

        @pl.loop(0, ch)
        def _(j):
            pltpu.sync_copy(ones_v, acc_sh.at[dst_v.at[j]], add=True)

        plsc.subcore_barrier()
        pltpu.sync_copy(acc_sh.at[pl.ds(r0, rows_per_tile)],
                        out_hbm.at[c, pl.ds(r0, rows_per_tile)])

    return count_kernel


def _make_agg_kernel(n, d, ch, k):
    rows_per_tile = n // NUM_TILES

    @functools.partial(
        pl.kernel,
        mesh=_sc_mesh(),
        out_type=jax.ShapeDtypeStruct((NUM_SC, n, d), jnp.float32),
        scratch_types=[
            pltpu.VMEM((ch, k), jnp.int32),
            pltpu.VMEM((ch, k), jnp.int32),
            pltpu.VMEM((k, d), jnp.float32),
            pltpu.VMEM_SHARED((n, d), jnp.float32),
        ],
    )
    def agg_kernel(h_hbm, src_hbm, dst_hbm, zero_hbm, out_hbm,
                   src_v, dst_v, rows_v, acc_sh):
        c = lax.axis_index("c")
        s = lax.axis_index("s")
        wid = s * NUM_SC + c
        r0 = s * rows_per_tile
        pltpu.sync_copy(zero_hbm.at[pl.ds(r0, rows_per_tile)],
                        acc_sh.at[pl.ds(r0, rows_per_tile)])
        pltpu.sync_copy(src_hbm.at[wid], src_v)
        pltpu.sync_copy(dst_hbm.at[wid], dst_v)
        plsc.subcore_barrier()

        @pl.loop(0, ch)
        def _(j):
            pltpu.sync_copy(h_hbm.at[src_v.at[j]], rows_v)
            pltpu.sync_copy(rows_v, acc_sh.at[dst_v.at[j]], add=True)

        plsc.subcore_barrier()
        pltpu.sync_copy(acc_sh.at[pl.ds(r0, rows_per_tile)],
                        out_hbm.at[c, pl.ds(r0, rows_per_tile)])

    return agg_kernel


def _tc_layer_body(p_ref, cp_ref, h_ref, wl_ref, wr_ref, b_ref, g_ref, be_ref,
                   o_ref, *, bn):
    n = h_ref.shape[0]
    cnt = cp_ref[0, :n, 0:1] + cp_ref[1, :n, 0:1]
    mean = (p_ref[0, :n, :] + p_ref[1, :n, :]) / jnp.maximum(cnt, 1.0)
    z = jnp.dot(mean, wl_ref[...], preferred_element_type=jnp.float32)
    z = z + jnp.dot(h_ref[...], wr_ref[...], preferred_element_type=jnp.float32)
    z = z + b_ref[...]
    if bn:
        mu = jnp.mean(z, axis=0, keepdims=True)
        var = jnp.mean((z - mu) ** 2, axis=0, keepdims=True)
        z = (z - mu) * lax.rsqrt(var + 1e-5) * g_ref[...] + be_ref[...]
        z = jnp.maximum(z, 0.0)
    o_ref[...] = z


def _tc_layer(p, cp, h, wl, wr, b, g, be, *, bn):
    n, d = h.shape
    body = functools.partial(_tc_layer_body, bn=bn)
    return pl.pallas_call(
        body,
        out_shape=jax.ShapeDtypeStruct((n, d), jnp.float32),
    )(p, cp, h, wl, wr, b.reshape(1, d), g.reshape(1, d), be.reshape(1, d))


def kernel(x, edge_index, Wl0, Wr0, b0, Wl1, Wr1, b1, Wl2, Wr2, b2,
           g0, be0, g1, be1):
    n, d = x.shape
    e = edge_index.shape[1]

    npad = -(-n // (NUM_TILES * 8)) * (NUM_TILES * 8)

    epw = e // NW
    k = 8
    for cand in range(8, 129, 8):
        if epw % cand == 0:
            k = cand
    ch = epw // k

    src3 = edge_index[0].astype(jnp.int32).reshape(NW, ch, k)
    dst3 = edge_index[1].astype(jnp.int32).reshape(NW, ch, k)
    zeros_nd = jnp.zeros((npad, d), jnp.float32)
    ones_k = jnp.ones((k, d), jnp.float32)

    count_k = _make_count_kernel(npad, d, ch, k)
    agg_k = _make_agg_kernel(npad, d, ch, k)

    cp = count_k(dst3, zeros_nd, ones_k)
    p0 = agg_k(x, src3, dst3, zeros_nd)
    h1 = _tc_layer(p0, cp, x, Wl0, Wr0, b0, g0, be0, bn=True)
    p1 = agg_k(h1, src3, dst3, zeros_nd)
    h2 = _tc_layer(p1, cp, h1, Wl1, Wr1, b1, g1, be1, bn=True)
    p2 = agg_k(h2, src3, dst3, zeros_nd)
    return _tc_layer(p2, cp, h2, Wl2, Wr2, b2, g1, be1, bn=False)


# speedup vs baseline: 2.1614x; 1.5301x over previous
import functools

import jax
import jax.numpy as jnp
from jax import lax
from jax.experimental import pallas as pl
from jax.experimental.pallas import tpu as pltpu
from jax.experimental.pallas import tpu_sc as plsc

NUM_SC = 2
NUM_TILES = 16
NW = NUM_SC * NUM_TILES


def _sc_mesh():
    return plsc.VectorSubcoreMesh(core_axis_name="c", subcore_axis_name="s")


def _make_count_kernel(n, d, ch, k):
    rows_per_tile = n // NUM_TILES
    win = 8

    @functools.partial(
        pl.kernel,
        mesh=_sc_mesh(),
        out_type=jax.ShapeDtypeStruct((NUM_SC, n, d), jnp.float32),
        scratch_types=[
            pltpu.VMEM((ch, k), jnp.int32),
            pltpu.VMEM((k, d), jnp.float32),
            pltpu.VMEM_SHARED((n, d), jnp.float32),
            pltpu.SemaphoreType.DMA,
        ],
    )
    def count_kernel(dst_hbm, zero_hbm, one_hbm, out_hbm, dst_v, ones_v,
                     acc_sh, sem):
        c = lax.axis_index("c")
        s = lax.axis_index("s")
        wid = s * NUM_SC + c
        r0 = s * rows_per_tile
        pltpu.sync_copy(zero_hbm.at[pl.ds(r0, rows_per_tile)],
                        acc_sh.at[pl.ds(r0, rows_per_tile)])
        pltpu.sync_copy(dst_hbm.at[wid], dst_v)
        pltpu.sync_copy(one_hbm, ones_v)
        plsc.subcore_barrier()

        @pl.loop(0, ch)
        def _(j):
            pltpu.sync_copy(ones_v, acc_sh.at[dst_v.at[j]], add=True)

        plsc.subcore_barrier()
        pltpu.sync_copy(acc_sh.at[pl.ds(r0, rows_per_tile)],
                        out_hbm.at[c, pl.ds(r0, rows_per_tile)])

    return count_kernel


def _make_agg_kernel(n, d, ch, k):
    rows_per_tile = n // NUM_TILES

    @functools.partial(
        pl.kernel,
        mesh=_sc_mesh(),
        out_type=jax.ShapeDtypeStruct((NUM_SC, n, d), jnp.float32),
        scratch_types=[
            pltpu.VMEM((ch, k), jnp.int32),
            pltpu.VMEM((k,), jnp.int32),
            pltpu.VMEM((k,), jnp.int32),
            pltpu.VMEM((k, d), jnp.float32),
            pltpu.VMEM((k, d), jnp.float32),
            pltpu.VMEM_SHARED((n, d), jnp.float32),
            pltpu.SemaphoreType.DMA,
            pltpu.SemaphoreType.DMA,
            pltpu.SemaphoreType.DMA,
            pltpu.SemaphoreType.DMA,
        ],
    )
    def agg_kernel(h_hbm, src_hbm, dst_hbm, zero_hbm, out_hbm,
                   src_v, db0, db1, rows0, rows1, acc_sh,
                   sg0, sg1, sd0, sd1):
        c = lax.axis_index("c")
        s = lax.axis_index("s")
        wid = s * NUM_SC + c
        r0 = s * rows_per_tile
        pltpu.sync_copy(zero_hbm.at[pl.ds(r0, rows_per_tile)],
                        acc_sh.at[pl.ds(r0, rows_per_tile)])
        pltpu.sync_copy(src_hbm.at[wid], src_v)
        plsc.subcore_barrier()

        dstb = (db0, db1)
        rows = (rows0, rows1)
        sg = (sg0, sg1)
        sd = (sd0, sd1)

        for b in range(2):
            pltpu.async_copy(dst_hbm.at[wid, b], dstb[b], sd[b])
            pltpu.async_copy(h_hbm.at[src_v.at[b]], rows[b], sg[b])

        @pl.loop(0, ch, step=2)
        def _(j):
            for b in range(2):
                jj = j + b

                @pl.when(jj < ch)
                def _():
                    pltpu.make_async_copy(h_hbm.at[src_v.at[jj]], rows[b],
                                          sg[b]).wait()
                    pltpu.make_async_copy(dst_hbm.at[wid, jj], dstb[b],
                                          sd[b]).wait()
                    pltpu.sync_copy(rows[b], acc_sh.at[dstb[b]], add=True)

                    @pl.when(jj + 2 < ch)
                    def _():
                        pltpu.async_copy(dst_hbm.at[wid, jj + 2], dstb[b],
                                         sd[b])
                        pltpu.async_copy(h_hbm.at[src_v.at[jj + 2]], rows[b],
                                         sg[b])

        plsc.subcore_barrier()
        pltpu.sync_copy(acc_sh.at[pl.ds(r0, rows_per_tile)],
                        out_hbm.at[c, pl.ds(r0, rows_per_tile)])

    return agg_kernel


def _tc_layer_body(p_ref, cp_ref, h_ref, wl_ref, wr_ref, b_ref, g_ref, be_ref,
                   o_ref, *, bn):
    n = h_ref.shape[0]
    cnt = cp_ref[0, :n, 0:1] + cp_ref[1, :n, 0:1]
    mean = (p_ref[0, :n, :] + p_ref[1, :n, :]) / jnp.maximum(cnt, 1.0)
    z = jnp.dot(mean, wl_ref[...], preferred_element_type=jnp.float32)
    z = z + jnp.dot(h_ref[...], wr_ref[...], preferred_element_type=jnp.float32)
    z = z + b_ref[...]
    if bn:
        mu = jnp.mean(z, axis=0, keepdims=True)
        var = jnp.mean((z - mu) ** 2, axis=0, keepdims=True)
        z = (z - mu) * lax.rsqrt(var + 1e-5) * g_ref[...] + be_ref[...]
        z = jnp.maximum(z, 0.0)
    o_ref[...] = z


def _tc_layer(p, cp, h, wl, wr, b, g, be, *, bn):
    n, d = h.shape
    body = functools.partial(_tc_layer_body, bn=bn)
    return pl.pallas_call(
        body,
        out_shape=jax.ShapeDtypeStruct((n, d), jnp.float32),
    )(p, cp, h, wl, wr, b.reshape(1, d), g.reshape(1, d), be.reshape(1, d))


def kernel(x, edge_index, Wl0, Wr0, b0, Wl1, Wr1, b1, Wl2, Wr2, b2,
           g0, be0, g1, be1):
    n, d = x.shape
    e = edge_index.shape[1]

    npad = -(-n // (NUM_TILES * 8)) * (NUM_TILES * 8)

    epw = e // NW
    k = 8
    for cand in range(8, 129, 8):
        if epw % cand == 0:
            k = cand
    ch = epw // k

    src3 = edge_index[0].astype(jnp.int32).reshape(NW, ch, k)
    dst3 = edge_index[1].astype(jnp.int32).reshape(NW, ch, k)
    zeros_nd = jnp.zeros((npad, d), jnp.float32)
    ones_k = jnp.ones((k, d), jnp.float32)

    count_k = _make_count_kernel(npad, d, ch, k)
    agg_k = _make_agg_kernel(npad, d, ch, k)

    cp = count_k(dst3, zeros_nd, ones_k)
    p0 = agg_k(x, src3, dst3, zeros_nd)
    h1 = _tc_layer(p0, cp, x, Wl0, Wr0, b0, g0, be0, bn=True)
    p1 = agg_k(h1, src3, dst3, zeros_nd)
    h2 = _tc_layer(p1, cp, h1, Wl1, Wr1, b1, g1, be1, bn=True)
    p2 = agg_k(h2, src3, dst3, zeros_nd)
    return _tc_layer(p2, cp, h2, Wl2, Wr2, b2, g1, be1, bn=False)
